# Initial kernel scaffold; baseline (speedup 1.0000x reference)
import jax
import jax.numpy as jnp
from jax.experimental import pallas as pl


def kernel(mag, phase, edge_index, rbf, W1m, b1m, W2m, b2m, W1p, b1p, W2p, b2p, phase_scale, Wg, bg, gamma, beta):
    raise NotImplementedError("write your pallas kernel here")


# speedup vs baseline: 1.9625x; 1.9625x over previous
import functools

import jax
import jax.numpy as jnp
from jax import lax
from jax.experimental import pallas as pl
from jax.experimental.pallas import tpu as pltpu
from jax.experimental.pallas import tpu_sc as plsc

N = 10000
E = 320000
D = 128
ED = 16

NC = 2
NS = 16
LANES = 16

EPT = E // NS
CHUNK = 80
NCHUNK = EPT // CHUNK
ROWS_MAIN = 624
ZROWS = 16


_BE = 4000


def _edge_mlp_body(rbf_ref, w1m_ref, b1m_ref, w2m_ref, b2m_ref,
                   w1p_ref, b1p_ref, w2p_ref, b2p_ref, ps_ref, out_ref):
    rbf = rbf_ref[...]
    hm = rbf @ w1m_ref[...] + b1m_ref[...]
    hm = hm * jax.nn.sigmoid(hm)
    em = jax.nn.softplus(hm @ w2m_ref[...] + b2m_ref[...])
    hp = rbf @ w1p_ref[...] + b1p_ref[...]
    hp = hp * jax.nn.sigmoid(hp)
    ep = jnp.tanh(hp @ w2p_ref[...] + b2p_ref[...]) * ps_ref[0, 0]
    out_ref[0] = em
    out_ref[1] = ep


def _edge_mlp(rbf, w1m, b1m, w2m, b2m, w1p, b1p, w2p, b2p, phase_scale):
    grid = E // _BE
    full = lambda shape: pl.BlockSpec(shape, lambda i: (0,) * len(shape))
    return pl.pallas_call(
        _edge_mlp_body,
        grid=(grid,),
        in_specs=[
            pl.BlockSpec((_BE, ED), lambda i: (i, 0)),
            full((ED, D)), full((1, D)), full((D, D)), full((1, D)),
            full((ED, D)), full((1, D)), full((D, D)), full((1, D)),
            full((1, 1)),
        ],
        out_specs=pl.BlockSpec((2, _BE, D), lambda i: (0, i, 0)),
        out_shape=jax.ShapeDtypeStruct((2, E, D), jnp.float32),
    )(rbf, w1m, b1m.reshape(1, D), w2m, b2m.reshape(1, D),
      w1p, b1p.reshape(1, D), w2p, b2p.reshape(1, D),
      phase_scale.reshape(1, 1))


def _sc_body(mag_hbm, phase_hbm, ev_hbm, src_hbm, dst_hbm,
             aggm_hbm, aggp_hbm,
             agg_sh, src_c, dst_c, gbuf, ebuf, zbuf, sem):
    c = lax.axis_index("c")
    s = lax.axis_index("s")

    zero = jnp.zeros((LANES,), jnp.float32)

    def zrow(r, _):
        for k in range(D // LANES):
            zbuf[r, pl.ds(k * LANES, LANES)] = zero
        return 0

    lax.fori_loop(0, ZROWS, zrow, 0)
    nz = jnp.where(s == NS - 1, ROWS_MAIN // ZROWS + 1, ROWS_MAIN // ZROWS)

    def zcopy(t, _):
        pltpu.sync_copy(zbuf, agg_sh.at[pl.ds(s * ROWS_MAIN + t * ZROWS,
                                              ZROWS)])
        return 0

    lax.fori_loop(0, nz, zcopy, 0)
    plsc.subcore_barrier()

    def run(table_hbm, ev_idx, combine):
        def chunk(j, _):
            base = s * EPT + j * CHUNK
            pltpu.sync_copy(src_hbm.at[s].at[j], src_c)
            pltpu.sync_copy(dst_hbm.at[s].at[j], dst_c)
            pltpu.sync_copy(ev_hbm.at[ev_idx].at[pl.ds(base, CHUNK)], ebuf)
            pltpu.async_copy(table_hbm.at[src_c], gbuf, sem).wait()

            def row(r, _):
                for k in range(D // LANES):
                    sl = pl.ds(k * LANES, LANES)
                    gbuf[r, sl] = combine(gbuf[r, sl], ebuf[r, sl])
                return 0

            lax.fori_loop(0, CHUNK, row, 0)
            pltpu.sync_copy(gbuf, agg_sh.at[dst_c], add=True)
            return 0

        lax.fori_loop(0, NCHUNK, chunk, 0)

    @pl.when(c == 0)
    def _():
        run(mag_hbm, 0, lambda g, e: g * e)

    @pl.when(c == 1)
    def _():
        run(phase_hbm, 1, lambda g, e: g + e)

    plsc.subcore_barrier()

    osl = pl.ds(s * ROWS_MAIN, ROWS_MAIN)
    tail = pl.ds(NS * ROWS_MAIN, N - NS * ROWS_MAIN)

    @pl.when(c == 0)
    def _():
        pltpu.sync_copy(agg_sh.at[osl], aggm_hbm.at[osl])

        @pl.when(s == NS - 1)
        def _():
            pltpu.sync_copy(agg_sh.at[tail], aggm_hbm.at[tail])

    @pl.when(c == 1)
    def _():
        pltpu.sync_copy(agg_sh.at[osl], aggp_hbm.at[osl])

        @pl.when(s == NS - 1)
        def _():
            pltpu.sync_copy(agg_sh.at[tail], aggp_hbm.at[tail])


def _sc_aggregate(mag, phase, ev, src2, dst2):
    mesh = plsc.VectorSubcoreMesh(core_axis_name="c", subcore_axis_name="s",
                                  num_cores=NC, num_subcores=NS)
    call = pl.kernel(
        _sc_body,
        out_type=(
            jax.ShapeDtypeStruct((N, D), jnp.float32),
            jax.ShapeDtypeStruct((N, D), jnp.float32),
        ),
        mesh=mesh,
        scratch_types=[
            pltpu.VMEM_SHARED((N, D), jnp.float32),
            pltpu.VMEM((CHUNK,), jnp.int32),
            pltpu.VMEM((CHUNK,), jnp.int32),
            pltpu.VMEM((CHUNK, D), jnp.float32),
            pltpu.VMEM((CHUNK, D), jnp.float32),
            pltpu.VMEM((ZROWS, D), jnp.float32),
            pltpu.SemaphoreType.DMA,
        ],
    )
    return call(mag, phase, ev, src2, dst2)


_BN = 2000


def _node_body(mag_ref, phase_ref, aggm_ref, aggp_ref,
               wg1_ref, wg2_ref, bg_ref, gamma_ref, beta_ref,
               outm_ref, outp_ref):
    mag = mag_ref[...]
    aggm = aggm_ref[...]
    gate = jax.nn.sigmoid(mag @ wg1_ref[...] + aggm @ wg2_ref[...]
                          + bg_ref[...])
    x = mag + gate * aggm
    mu = jnp.mean(x, axis=-1, keepdims=True)
    xc = x - mu
    var = jnp.mean(xc * xc, axis=-1, keepdims=True)
    outm_ref[...] = xc * lax.rsqrt(var + 1e-5) * gamma_ref[...] + beta_ref[...]
    outp_ref[...] = phase_ref[...] + gate * aggp_ref[...]


def _node_update(mag, phase, aggm, aggp, wg, bg, gamma, beta):
    grid = N // _BN
    full = lambda shape: pl.BlockSpec(shape, lambda i: (0,) * len(shape))
    blk = pl.BlockSpec((_BN, D), lambda i: (i, 0))
    return pl.pallas_call(
        _node_body,
        grid=(grid,),
        in_specs=[blk, blk, blk, blk,
                  full((D, D)), full((D, D)),
                  full((1, D)), full((1, D)), full((1, D))],
        out_specs=(blk, blk),
        out_shape=(jax.ShapeDtypeStruct((N, D), jnp.float32),
                   jax.ShapeDtypeStruct((N, D), jnp.float32)),
    )(mag, phase, aggm, aggp, wg[:D], wg[D:],
      bg.reshape(1, D), gamma.reshape(1, D), beta.reshape(1, D))


def kernel(mag, phase, edge_index, rbf, W1m, b1m, W2m, b2m,
           W1p, b1p, W2p, b2p, phase_scale, Wg, bg, gamma, beta):
    ev = _edge_mlp(rbf, W1m, b1m, W2m, b2m, W1p, b1p, W2p, b2p, phase_scale)
    src2 = edge_index[0].reshape(NS, NCHUNK, CHUNK)
    dst2 = edge_index[1].reshape(NS, NCHUNK, CHUNK)
    aggm, aggp = _sc_aggregate(mag, phase, ev, src2, dst2)
    return _node_update(mag, phase, aggm, aggp, Wg, bg, gamma, beta)


# speedup vs baseline: 3.1011x; 1.5802x over previous
import jax
import jax.numpy as jnp
from jax import lax
from jax.experimental import pallas as pl
from jax.experimental.pallas import tpu as pltpu
from jax.experimental.pallas import tpu_sc as plsc

N = 10000
E = 320000
D = 128
ED = 16

NC = 2
NS = 16
LANES = 16

CHUNK = 80
G = 10
NG = 26
NGP = NG // 2
EPT = NG * G * CHUNK
E2 = NS * EPT
PAD_ROWS = 16
NAGG = N + PAD_ROWS
ROWS_MAIN = 624


_BE = 3200


def _edge_mlp_body(rbf_ref, w1m_ref, b1m_ref, w2m_ref, b2m_ref,
                   w1p_ref, b1p_ref, w2p_ref, b2p_ref, ps_ref, out_ref):
    rbf = rbf_ref[...]
    hm = rbf @ w1m_ref[...] + b1m_ref[...]
    hm = hm * jax.nn.sigmoid(hm)
    em = jax.nn.softplus(hm @ w2m_ref[...] + b2m_ref[...])
    hp = rbf @ w1p_ref[...] + b1p_ref[...]
    hp = hp * jax.nn.sigmoid(hp)
    ep = jnp.tanh(hp @ w2p_ref[...] + b2p_ref[...]) * ps_ref[0, 0]
    out_ref[0] = em
    out_ref[1] = ep


def _edge_mlp(rbf, w1m, b1m, w2m, b2m, w1p, b1p, w2p, b2p, phase_scale):
    grid = E2 // _BE
    full = lambda shape: pl.BlockSpec(shape, lambda i: (0,) * len(shape))
    return pl.pallas_call(
        _edge_mlp_body,
        grid=(grid,),
        in_specs=[
            pl.BlockSpec((_BE, ED), lambda i: (i, 0)),
            full((ED, D)), full((1, D)), full((D, D)), full((1, D)),
            full((ED, D)), full((1, D)), full((D, D)), full((1, D)),
            full((1, 1)),
        ],
        out_specs=pl.BlockSpec((2, _BE, D), lambda i: (0, i, 0)),
        out_shape=jax.ShapeDtypeStruct((2, E2, D), jnp.float32),
    )(rbf, w1m, b1m.reshape(1, D), w2m, b2m.reshape(1, D),
      w1p, b1p.reshape(1, D), w2p, b2p.reshape(1, D),
      phase_scale.reshape(1, 1))


def _sc_body(mag_hbm, phase_hbm, ev_hbm, srcg_hbm, dstg_hbm,
             aggm_hbm, aggp_hbm,
             agg_sh, sidx0, sidx1, didx0, didx1, gb0, gb1, eb0, eb1,
             sg0, sg1, se0, se1, ss0, ss1, si0, si1):
    c = lax.axis_index("c")
    s = lax.axis_index("s")
    sidx = [sidx0, sidx1]
    didx = [didx0, didx1]
    gb = [gb0, gb1]
    eb = [eb0, eb1]
    sg = [sg0, sg1]
    se = [se0, se1]
    ss = [ss0, ss1]
    si = [si0, si1]

    zero = jnp.zeros((LANES,), jnp.float32)

    def zrow(r, _):
        for m in range(D // LANES):
            gb0[r, pl.ds(m * LANES, LANES)] = zero
        return 0

    lax.fori_loop(0, CHUNK, zrow, 0)

    def zcopy(t, _):
        pltpu.sync_copy(gb0, agg_sh.at[pl.ds(s * ROWS_MAIN + t * CHUNK,
                                             CHUNK)])
        return 0

    lax.fori_loop(0, 7, zcopy, 0)

    @pl.when(s < NS - 1)
    def _():
        pltpu.sync_copy(gb0.at[pl.ds(0, 64)],
                        agg_sh.at[pl.ds(s * ROWS_MAIN + 560, 64)])

    @pl.when(s == NS - 1)
    def _():
        pltpu.sync_copy(gb0, agg_sh.at[pl.ds(9920, CHUNK)])
        pltpu.sync_copy(gb0.at[pl.ds(0, PAD_ROWS)],
                        agg_sh.at[pl.ds(N, PAD_ROWS)])

    plsc.subcore_barrier()

    def run(table_hbm, ci, combine):
        def wait_gather(q):
            pltpu.make_async_copy(table_hbm.at[sidx0.at[0]], gb[q],
                                  sg[q]).wait()

        def wait_ebuf(q):
            pltpu.make_async_copy(ev_hbm.at[ci].at[pl.ds(0, CHUNK)], eb[q],
                                  se[q]).wait()

        def wait_scat(q):
            pltpu.make_async_copy(eb[q], agg_sh.at[didx0.at[0]],
                                  ss[q]).wait()

        def wait_slab(p):
            pltpu.make_async_copy(srcg_hbm.at[s].at[0], sidx[p],
                                  si[p]).wait()
            pltpu.make_async_copy(dstg_hbm.at[s].at[0], didx[p],
                                  si[p]).wait()

        pltpu.sync_copy(srcg_hbm.at[s].at[0], sidx[0])
        pltpu.sync_copy(dstg_hbm.at[s].at[0], didx[0])
        pltpu.async_copy(srcg_hbm.at[s].at[1], sidx[1], si[1])
        pltpu.async_copy(dstg_hbm.at[s].at[1], didx[1], si[1])
        pltpu.async_copy(ev_hbm.at[ci].at[pl.ds(s * EPT, CHUNK)], eb[0],
                         se[0])
        pltpu.async_copy(table_hbm.at[sidx[0].at[0]], gb[0], sg[0])

        def pair(gg, _):
            for pg in (0, 1):
                g = gg * 2 + pg
                for k in range(G):
                    j = g * G + k
                    q = k & 1
                    qn = q ^ 1
                    pnext = pg ^ 1
                    islast = (pg == 1 and k == G - 1)
                    wait_ebuf(q)
                    wait_gather(q)
                    if k == G - 1:
                        def _issue_g():
                            wait_slab(pnext)
                            pltpu.async_copy(
                                table_hbm.at[sidx[pnext].at[0]], gb[qn],
                                sg[qn])
                        if islast:
                            pl.when(gg < NGP - 1)(_issue_g)
                        else:
                            _issue_g()
                    else:
                        pltpu.async_copy(table_hbm.at[sidx[pg].at[k + 1]],
                                         gb[qn], sg[qn])
                    def crow(r, _):
                        for m in range(D // LANES):
                            sl = pl.ds(m * LANES, LANES)
                            eb[q][r, sl] = combine(gb[q][r, sl],
                                                   eb[q][r, sl])
                        return 0

                    lax.fori_loop(0, CHUNK, crow, 0)
                    if pg == 0 and k == 0:
                        pl.when(gg > 0)(lambda: wait_scat(qn))
                    else:
                        wait_scat(qn)

                    def _issue_e():
                        pltpu.async_copy(
                            ev_hbm.at[ci].at[
                                pl.ds(s * EPT + (j + 1) * CHUNK, CHUNK)],
                            eb[qn], se[qn])
                    if islast:
                        pl.when(gg < NGP - 1)(_issue_e)
                    else:
                        _issue_e()
                    pltpu.async_copy(eb[q], agg_sh.at[didx[pg].at[k]],
                                     ss[q], add=True)
                    if k == 2:
                        def _issue_slab():
                            pltpu.async_copy(srcg_hbm.at[s].at[g + 1],
                                             sidx[pnext], si[pnext])
                            pltpu.async_copy(dstg_hbm.at[s].at[g + 1],
                                             didx[pnext], si[pnext])
                        pl.when((g >= 1) & (g < NG - 1))(_issue_slab)
            return 0

        lax.fori_loop(0, NGP, pair, 0)
        wait_scat(1)

    @pl.when(c == 0)
    def _():
        run(mag_hbm, 0, lambda g_, e_: g_ * e_)

    @pl.when(c == 1)
    def _():
        run(phase_hbm, 1, lambda g_, e_: g_ + e_)

    plsc.subcore_barrier()

    osl = pl.ds(s * ROWS_MAIN, ROWS_MAIN)
    tail = pl.ds(NS * ROWS_MAIN, N - NS * ROWS_MAIN)

    @pl.when(c == 0)
    def _():
        pltpu.sync_copy(agg_sh.at[osl], aggm_hbm.at[osl])

        @pl.when(s == NS - 1)
        def _():
            pltpu.sync_copy(agg_sh.at[tail], aggm_hbm.at[tail])

    @pl.when(c == 1)
    def _():
        pltpu.sync_copy(agg_sh.at[osl], aggp_hbm.at[osl])

        @pl.when(s == NS - 1)
        def _():
            pltpu.sync_copy(agg_sh.at[tail], aggp_hbm.at[tail])


def _sc_aggregate(mag, phase, ev, srcg, dstg):
    mesh = plsc.VectorSubcoreMesh(core_axis_name="c", subcore_axis_name="s",
                                  num_cores=NC, num_subcores=NS)
    call = pl.kernel(
        _sc_body,
        out_type=(
            jax.ShapeDtypeStruct((N, D), jnp.float32),
            jax.ShapeDtypeStruct((N, D), jnp.float32),
        ),
        mesh=mesh,
        scratch_types=[
            pltpu.VMEM_SHARED((NAGG, D), jnp.float32),
            pltpu.VMEM((G, CHUNK), jnp.int32),
            pltpu.VMEM((G, CHUNK), jnp.int32),
            pltpu.VMEM((G, CHUNK), jnp.int32),
            pltpu.VMEM((G, CHUNK), jnp.int32),
            pltpu.VMEM((CHUNK, D), jnp.float32),
            pltpu.VMEM((CHUNK, D), jnp.float32),
            pltpu.VMEM((CHUNK, D), jnp.float32),
            pltpu.VMEM((CHUNK, D), jnp.float32),
            pltpu.SemaphoreType.DMA,
            pltpu.SemaphoreType.DMA,
            pltpu.SemaphoreType.DMA,
            pltpu.SemaphoreType.DMA,
            pltpu.SemaphoreType.DMA,
            pltpu.SemaphoreType.DMA,
            pltpu.SemaphoreType.DMA,
            pltpu.SemaphoreType.DMA,
        ],
    )
    return call(mag, phase, ev, srcg, dstg)


_BN = 2000


def _node_body(mag_ref, phase_ref, aggm_ref, aggp_ref,
               wg1_ref, wg2_ref, bg_ref, gamma_ref, beta_ref,
               outm_ref, outp_ref):
    mag = mag_ref[...]
    aggm = aggm_ref[...]
    gate = jax.nn.sigmoid(mag @ wg1_ref[...] + aggm @ wg2_ref[...]
                          + bg_ref[...])
    x = mag + gate * aggm
    mu = jnp.mean(x, axis=-1, keepdims=True)
    xc = x - mu
    var = jnp.mean(xc * xc, axis=-1, keepdims=True)
    outm_ref[...] = xc * lax.rsqrt(var + 1e-5) * gamma_ref[...] + beta_ref[...]
    outp_ref[...] = phase_ref[...] + gate * aggp_ref[...]


def _node_update(mag, phase, aggm, aggp, wg, bg, gamma, beta):
    grid = N // _BN
    full = lambda shape: pl.BlockSpec(shape, lambda i: (0,) * len(shape))
    blk = pl.BlockSpec((_BN, D), lambda i: (i, 0))
    return pl.pallas_call(
        _node_body,
        grid=(grid,),
        in_specs=[blk, blk, blk, blk,
                  full((D, D)), full((D, D)),
                  full((1, D)), full((1, D)), full((1, D))],
        out_specs=(blk, blk),
        out_shape=(jax.ShapeDtypeStruct((N, D), jnp.float32),
                   jax.ShapeDtypeStruct((N, D), jnp.float32)),
    )(mag, phase, aggm, aggp, wg[:D], wg[D:],
      bg.reshape(1, D), gamma.reshape(1, D), beta.reshape(1, D))


def kernel(mag, phase, edge_index, rbf, W1m, b1m, W2m, b2m,
           W1p, b1p, W2p, b2p, phase_scale, Wg, bg, gamma, beta):
    pad_n = E2 - E
    pe = jnp.arange(pad_n, dtype=jnp.int32)
    src_p = jnp.concatenate([edge_index[0], (pe * 37) % N])
    dst_p = jnp.concatenate([edge_index[1], N + (pe % PAD_ROWS)])
    srcg = src_p.reshape(NS, NG, G, CHUNK)
    dstg = dst_p.reshape(NS, NG, G, CHUNK)
    rbf_p = jnp.concatenate(
        [rbf, jnp.zeros((pad_n, ED), dtype=rbf.dtype)])
    ev = _edge_mlp(rbf_p, W1m, b1m, W2m, b2m, W1p, b1p, W2p, b2p,
                   phase_scale)
    aggm, aggp = _sc_aggregate(mag, phase, ev, srcg, dstg)
    return _node_update(mag, phase, aggm, aggp, Wg, bg, gamma, beta)


# speedup vs baseline: 3.3996x; 1.0963x over previous
import jax
import jax.numpy as jnp
from jax import lax
from jax.experimental import pallas as pl
from jax.experimental.pallas import tpu as pltpu
from jax.experimental.pallas import tpu_sc as plsc

N = 10000
E = 320000
D = 128
ED = 16

NC = 2
NS = 16
LANES = 16

CHUNK = 80
G = 10
NG = 26
NGP = NG // 2
EPT = NG * G * CHUNK
E2 = NS * EPT
PAD_ROWS = 16
NAGG = N + PAD_ROWS
ROWS_MAIN = 624


_BE = 3200


def _edge_mlp_body(rbf_ref, w1m_ref, b1m_ref, w2m_ref, b2m_ref,
                   w1p_ref, b1p_ref, w2p_ref, b2p_ref, ps_ref, out_ref):
    rbf = rbf_ref[...]
    hm = rbf @ w1m_ref[...] + b1m_ref[...]
    hm = hm * jax.nn.sigmoid(hm)
    em = jax.nn.softplus(hm @ w2m_ref[...] + b2m_ref[...])
    hp = rbf @ w1p_ref[...] + b1p_ref[...]
    hp = hp * jax.nn.sigmoid(hp)
    ep = jnp.tanh(hp @ w2p_ref[...] + b2p_ref[...]) * ps_ref[0, 0]
    out_ref[0] = em
    out_ref[1] = ep


def _edge_mlp(rbf, w1m, b1m, w2m, b2m, w1p, b1p, w2p, b2p, phase_scale):
    grid = E2 // _BE
    full = lambda shape: pl.BlockSpec(shape, lambda i: (0,) * len(shape))
    return pl.pallas_call(
        _edge_mlp_body,
        grid=(grid,),
        in_specs=[
            pl.BlockSpec((_BE, ED), lambda i: (i, 0)),
            full((ED, D)), full((1, D)), full((D, D)), full((1, D)),
            full((ED, D)), full((1, D)), full((D, D)), full((1, D)),
            full((1, 1)),
        ],
        out_specs=pl.BlockSpec((2, _BE, D), lambda i: (0, i, 0)),
        out_shape=jax.ShapeDtypeStruct((2, E2, D), jnp.float32),
    )(rbf, w1m, b1m.reshape(1, D), w2m, b2m.reshape(1, D),
      w1p, b1p.reshape(1, D), w2p, b2p.reshape(1, D),
      phase_scale.reshape(1, 1))


def _sc_body(mag_hbm, phase_hbm, ev_hbm, srcg_hbm, dstg_hbm,
             aggm_hbm, aggp_hbm,
             agg_sh, sidx0, sidx1, didx0, didx1, gb0, gb1, eb0, eb1,
             sg0, sg1, se0, se1, ss0, ss1, si0, si1):
    c = lax.axis_index("c")
    s = lax.axis_index("s")
    sidx = [sidx0, sidx1]
    didx = [didx0, didx1]
    gb = [gb0, gb1]
    eb = [eb0, eb1]
    sg = [sg0, sg1]
    se = [se0, se1]
    ss = [ss0, ss1]
    si = [si0, si1]

    zero = jnp.zeros((LANES,), jnp.float32)

    def zrow(r, _):
        for m in range(D // LANES):
            gb0[r, pl.ds(m * LANES, LANES)] = zero
        return 0

    lax.fori_loop(0, CHUNK, zrow, 0)

    def zcopy(t, _):
        pltpu.sync_copy(gb0, agg_sh.at[pl.ds(s * ROWS_MAIN + t * CHUNK,
                                             CHUNK)])
        return 0

    lax.fori_loop(0, 7, zcopy, 0)

    @pl.when(s < NS - 1)
    def _():
        pltpu.sync_copy(gb0.at[pl.ds(0, 64)],
                        agg_sh.at[pl.ds(s * ROWS_MAIN + 560, 64)])

    @pl.when(s == NS - 1)
    def _():
        pltpu.sync_copy(gb0, agg_sh.at[pl.ds(9920, CHUNK)])
        pltpu.sync_copy(gb0.at[pl.ds(0, PAD_ROWS)],
                        agg_sh.at[pl.ds(N, PAD_ROWS)])

    plsc.subcore_barrier()

    def run(table_hbm, ci, combine):
        def wait_gather(q):
            pltpu.make_async_copy(table_hbm.at[sidx0.at[0]], gb[q],
                                  sg[q]).wait()

        def wait_ebuf(q):
            pltpu.make_async_copy(ev_hbm.at[ci].at[pl.ds(0, CHUNK)], eb[q],
                                  se[q]).wait()

        def wait_scat(q):
            pltpu.make_async_copy(eb[q], agg_sh.at[didx0.at[0]],
                                  ss[q]).wait()

        def wait_slab(p):
            pltpu.make_async_copy(srcg_hbm.at[s].at[0], sidx[p],
                                  si[p]).wait()
            pltpu.make_async_copy(dstg_hbm.at[s].at[0], didx[p],
                                  si[p]).wait()

        pltpu.sync_copy(srcg_hbm.at[s].at[0], sidx[0])
        pltpu.sync_copy(dstg_hbm.at[s].at[0], didx[0])
        pltpu.async_copy(srcg_hbm.at[s].at[1], sidx[1], si[1])
        pltpu.async_copy(dstg_hbm.at[s].at[1], didx[1], si[1])
        pltpu.async_copy(ev_hbm.at[ci].at[pl.ds(s * EPT, CHUNK)], eb[0],
                         se[0])
        pltpu.async_copy(table_hbm.at[sidx[0].at[0]], gb[0], sg[0])

        def pair(gg, _):
            for pg in (0, 1):
                g = gg * 2 + pg
                for k in range(G):
                    j = g * G + k
                    q = k & 1
                    qn = q ^ 1
                    pnext = pg ^ 1
                    islast = (pg == 1 and k == G - 1)
                    wait_ebuf(q)
                    wait_gather(q)
                    if k == G - 1:
                        def _issue_g():
                            wait_slab(pnext)
                            pltpu.async_copy(
                                table_hbm.at[sidx[pnext].at[0]], gb[qn],
                                sg[qn])
                        if islast:
                            pl.when(gg < NGP - 1)(_issue_g)
                        else:
                            _issue_g()
                    else:
                        pltpu.async_copy(table_hbm.at[sidx[pg].at[k + 1]],
                                         gb[qn], sg[qn])
                    if pg == 0 and k == 0:
                        pl.when(gg > 0)(lambda: wait_scat(qn))
                    else:
                        wait_scat(qn)

                    def _issue_e():
                        pltpu.async_copy(
                            ev_hbm.at[ci].at[
                                pl.ds(s * EPT + (j + 1) * CHUNK, CHUNK)],
                            eb[qn], se[qn])
                    if islast:
                        pl.when(gg < NGP - 1)(_issue_e)
                    else:
                        _issue_e()
                    def crow(r, _):
                        for rr in range(2):
                            for m in range(D // LANES):
                                sl = pl.ds(m * LANES, LANES)
                                row = 2 * r + rr
                                eb[q][row, sl] = combine(gb[q][row, sl],
                                                         eb[q][row, sl])
                        return 0

                    lax.fori_loop(0, CHUNK // 2, crow, 0)
                    pltpu.async_copy(eb[q], agg_sh.at[didx[pg].at[k]],
                                     ss[q], add=True)
                    if k == 2:
                        def _issue_slab():
                            pltpu.async_copy(srcg_hbm.at[s].at[g + 1],
                                             sidx[pnext], si[pnext])
                            pltpu.async_copy(dstg_hbm.at[s].at[g + 1],
                                             didx[pnext], si[pnext])
                        pl.when((g >= 1) & (g < NG - 1))(_issue_slab)
            return 0

        lax.fori_loop(0, NGP, pair, 0)
        wait_scat(1)

    @pl.when(c == 0)
    def _():
        run(mag_hbm, 0, lambda g_, e_: g_ * e_)

    @pl.when(c == 1)
    def _():
        run(phase_hbm, 1, lambda g_, e_: g_ + e_)

    plsc.subcore_barrier()

    osl = pl.ds(s * ROWS_MAIN, ROWS_MAIN)
    tail = pl.ds(NS * ROWS_MAIN, N - NS * ROWS_MAIN)

    @pl.when(c == 0)
    def _():
        pltpu.sync_copy(agg_sh.at[osl], aggm_hbm.at[osl])

        @pl.when(s == NS - 1)
        def _():
            pltpu.sync_copy(agg_sh.at[tail], aggm_hbm.at[tail])

    @pl.when(c == 1)
    def _():
        pltpu.sync_copy(agg_sh.at[osl], aggp_hbm.at[osl])

        @pl.when(s == NS - 1)
        def _():
            pltpu.sync_copy(agg_sh.at[tail], aggp_hbm.at[tail])


def _sc_aggregate(mag, phase, ev, srcg, dstg):
    mesh = plsc.VectorSubcoreMesh(core_axis_name="c", subcore_axis_name="s",
                                  num_cores=NC, num_subcores=NS)
    call = pl.kernel(
        _sc_body,
        out_type=(
            jax.ShapeDtypeStruct((N, D), jnp.float32),
            jax.ShapeDtypeStruct((N, D), jnp.float32),
        ),
        mesh=mesh,
        scratch_types=[
            pltpu.VMEM_SHARED((NAGG, D), jnp.float32),
            pltpu.VMEM((G, CHUNK), jnp.int32),
            pltpu.VMEM((G, CHUNK), jnp.int32),
            pltpu.VMEM((G, CHUNK), jnp.int32),
            pltpu.VMEM((G, CHUNK), jnp.int32),
            pltpu.VMEM((CHUNK, D), jnp.float32),
            pltpu.VMEM((CHUNK, D), jnp.float32),
            pltpu.VMEM((CHUNK, D), jnp.float32),
            pltpu.VMEM((CHUNK, D), jnp.float32),
            pltpu.SemaphoreType.DMA,
            pltpu.SemaphoreType.DMA,
            pltpu.SemaphoreType.DMA,
            pltpu.SemaphoreType.DMA,
            pltpu.SemaphoreType.DMA,
            pltpu.SemaphoreType.DMA,
            pltpu.SemaphoreType.DMA,
            pltpu.SemaphoreType.DMA,
        ],
    )
    return call(mag, phase, ev, srcg, dstg)


_BN = 2000


def _node_body(mag_ref, phase_ref, aggm_ref, aggp_ref,
               wg1_ref, wg2_ref, bg_ref, gamma_ref, beta_ref,
               outm_ref, outp_ref):
    mag = mag_ref[...]
    aggm = aggm_ref[...]
    gate = jax.nn.sigmoid(mag @ wg1_ref[...] + aggm @ wg2_ref[...]
                          + bg_ref[...])
    x = mag + gate * aggm
    mu = jnp.mean(x, axis=-1, keepdims=True)
    xc = x - mu
    var = jnp.mean(xc * xc, axis=-1, keepdims=True)
    outm_ref[...] = xc * lax.rsqrt(var + 1e-5) * gamma_ref[...] + beta_ref[...]
    outp_ref[...] = phase_ref[...] + gate * aggp_ref[...]


def _node_update(mag, phase, aggm, aggp, wg, bg, gamma, beta):
    grid = N // _BN
    full = lambda shape: pl.BlockSpec(shape, lambda i: (0,) * len(shape))
    blk = pl.BlockSpec((_BN, D), lambda i: (i, 0))
    return pl.pallas_call(
        _node_body,
        grid=(grid,),
        in_specs=[blk, blk, blk, blk,
                  full((D, D)), full((D, D)),
                  full((1, D)), full((1, D)), full((1, D))],
        out_specs=(blk, blk),
        out_shape=(jax.ShapeDtypeStruct((N, D), jnp.float32),
                   jax.ShapeDtypeStruct((N, D), jnp.float32)),
    )(mag, phase, aggm, aggp, wg[:D], wg[D:],
      bg.reshape(1, D), gamma.reshape(1, D), beta.reshape(1, D))


def kernel(mag, phase, edge_index, rbf, W1m, b1m, W2m, b2m,
           W1p, b1p, W2p, b2p, phase_scale, Wg, bg, gamma, beta):
    pad_n = E2 - E
    pe = jnp.arange(pad_n, dtype=jnp.int32)
    src_p = jnp.concatenate([edge_index[0], (pe * 37) % N])
    dst_p = jnp.concatenate([edge_index[1], N + (pe % PAD_ROWS)])
    srcg = src_p.reshape(NS, NG, G, CHUNK)
    dstg = dst_p.reshape(NS, NG, G, CHUNK)
    rbf_p = jnp.concatenate(
        [rbf, jnp.zeros((pad_n, ED), dtype=rbf.dtype)])
    ev = _edge_mlp(rbf_p, W1m, b1m, W2m, b2m, W1p, b1p, W2p, b2p,
                   phase_scale)
    aggm, aggp = _sc_aggregate(mag, phase, ev, srcg, dstg)
    return _node_update(mag, phase, aggm, aggp, Wg, bg, gamma, beta)


# speedup vs baseline: 3.5744x; 1.0514x over previous
import jax
import jax.numpy as jnp
from jax import lax
from jax.experimental import pallas as pl
from jax.experimental.pallas import tpu as pltpu
from jax.experimental.pallas import tpu_sc as plsc

N = 10000
E = 320000
D = 128
ED = 16

NC = 2
NS = 16
LANES = 16

CHUNK = 80
G = 10
NG = 26
NGP = NG // 2
EPT = NG * G * CHUNK
E2 = NS * EPT
PAD_ROWS = 16
NAGG = N + PAD_ROWS
ROWS_MAIN = 624


_BE = 3200


def _edge_mlp_body(rbf_ref, w1m_ref, b1m_ref, w2m_ref, b2m_ref,
                   w1p_ref, b1p_ref, w2p_ref, b2p_ref, ps_ref, out_ref):
    rbf = rbf_ref[...]
    hm = rbf @ w1m_ref[...] + b1m_ref[...]
    hm = hm * jax.nn.sigmoid(hm)
    em = jax.nn.softplus(hm @ w2m_ref[...] + b2m_ref[...])
    hp = rbf @ w1p_ref[...] + b1p_ref[...]
    hp = hp * jax.nn.sigmoid(hp)
    ep = jnp.tanh(hp @ w2p_ref[...] + b2p_ref[...]) * ps_ref[0, 0]
    out_ref[0] = em
    out_ref[1] = ep


def _edge_mlp(rbf, w1m, b1m, w2m, b2m, w1p, b1p, w2p, b2p, phase_scale):
    grid = E // _BE
    full = lambda shape: pl.BlockSpec(shape, lambda i: (0,) * len(shape))
    return pl.pallas_call(
        _edge_mlp_body,
        grid=(grid,),
        in_specs=[
            pl.BlockSpec((_BE, ED), lambda i: (i, 0)),
            full((ED, D)), full((1, D)), full((D, D)), full((1, D)),
            full((ED, D)), full((1, D)), full((D, D)), full((1, D)),
            full((1, 1)),
        ],
        out_specs=pl.BlockSpec((2, _BE, D), lambda i: (0, i, 0)),
        out_shape=jax.ShapeDtypeStruct((2, E2, D), jnp.float32),
    )(rbf, w1m, b1m.reshape(1, D), w2m, b2m.reshape(1, D),
      w1p, b1p.reshape(1, D), w2p, b2p.reshape(1, D),
      phase_scale.reshape(1, 1))


def _sc_body(mag_hbm, phase_hbm, ev_hbm, srcg_hbm, dstg_hbm,
             aggm_hbm, aggp_hbm,
             agg_sh, sidx0, sidx1, didx0, didx1, gb0, gb1, eb0, eb1,
             sg0, sg1, se0, se1, ss0, ss1, si0, si1):
    c = lax.axis_index("c")
    s = lax.axis_index("s")
    sidx = [sidx0, sidx1]
    didx = [didx0, didx1]
    gb = [gb0, gb1]
    eb = [eb0, eb1]
    sg = [sg0, sg1]
    se = [se0, se1]
    ss = [ss0, ss1]
    si = [si0, si1]

    zero = jnp.zeros((LANES,), jnp.float32)

    def zrow(r, _):
        for m in range(D // LANES):
            gb0[r, pl.ds(m * LANES, LANES)] = zero
        return 0

    lax.fori_loop(0, CHUNK, zrow, 0)

    def zcopy(t, _):
        pltpu.sync_copy(gb0, agg_sh.at[pl.ds(s * ROWS_MAIN + t * CHUNK,
                                             CHUNK)])
        return 0

    lax.fori_loop(0, 7, zcopy, 0)

    @pl.when(s < NS - 1)
    def _():
        pltpu.sync_copy(gb0.at[pl.ds(0, 64)],
                        agg_sh.at[pl.ds(s * ROWS_MAIN + 560, 64)])

    @pl.when(s == NS - 1)
    def _():
        pltpu.sync_copy(gb0, agg_sh.at[pl.ds(9920, CHUNK)])
        pltpu.sync_copy(gb0.at[pl.ds(0, PAD_ROWS)],
                        agg_sh.at[pl.ds(N, PAD_ROWS)])

    plsc.subcore_barrier()

    def run(table_hbm, ci, combine):
        def wait_gather(q):
            pltpu.make_async_copy(table_hbm.at[sidx0.at[0]], gb[q],
                                  sg[q]).wait()

        def wait_ebuf(q):
            pltpu.make_async_copy(ev_hbm.at[ci].at[pl.ds(0, CHUNK)], eb[q],
                                  se[q]).wait()

        def wait_scat(q):
            pltpu.make_async_copy(eb[q], agg_sh.at[didx0.at[0]],
                                  ss[q]).wait()

        def wait_slab(p):
            pltpu.make_async_copy(srcg_hbm.at[s].at[0], sidx[p],
                                  si[p]).wait()
            pltpu.make_async_copy(dstg_hbm.at[s].at[0], didx[p],
                                  si[p]).wait()

        pltpu.sync_copy(srcg_hbm.at[s].at[0], sidx[0])
        pltpu.sync_copy(dstg_hbm.at[s].at[0], didx[0])
        pltpu.async_copy(srcg_hbm.at[s].at[1], sidx[1], si[1])
        pltpu.async_copy(dstg_hbm.at[s].at[1], didx[1], si[1])
        pltpu.async_copy(ev_hbm.at[ci].at[pl.ds(s * EPT, CHUNK)], eb[0],
                         se[0])
        pltpu.async_copy(table_hbm.at[sidx[0].at[0]], gb[0], sg[0])

        def pair(gg, _):
            for pg in (0, 1):
                g = gg * 2 + pg
                for k in range(G):
                    j = g * G + k
                    q = k & 1
                    qn = q ^ 1
                    pnext = pg ^ 1
                    islast = (pg == 1 and k == G - 1)
                    wait_ebuf(q)
                    wait_gather(q)
                    if k == G - 1:
                        def _issue_g():
                            wait_slab(pnext)
                            pltpu.async_copy(
                                table_hbm.at[sidx[pnext].at[0]], gb[qn],
                                sg[qn])
                        if islast:
                            pl.when(gg < NGP - 1)(_issue_g)
                        else:
                            _issue_g()
                    else:
                        pltpu.async_copy(table_hbm.at[sidx[pg].at[k + 1]],
                                         gb[qn], sg[qn])
                    if pg == 0 and k == 0:
                        pl.when(gg > 0)(lambda: wait_scat(qn))
                    else:
                        wait_scat(qn)

                    def _issue_e():
                        pltpu.async_copy(
                            ev_hbm.at[ci].at[
                                pl.ds(s * EPT + (j + 1) * CHUNK, CHUNK)],
                            eb[qn], se[qn])
                    if islast:
                        pl.when(gg < NGP - 1)(_issue_e)
                    else:
                        _issue_e()
                    def crow(r, _):
                        for rr in range(2):
                            for m in range(D // LANES):
                                sl = pl.ds(m * LANES, LANES)
                                row = 2 * r + rr
                                eb[q][row, sl] = combine(gb[q][row, sl],
                                                         eb[q][row, sl])
                        return 0

                    lax.fori_loop(0, CHUNK // 2, crow, 0)
                    pltpu.async_copy(eb[q], agg_sh.at[didx[pg].at[k]],
                                     ss[q], add=True)
                    if k == 2:
                        def _issue_slab():
                            pltpu.async_copy(srcg_hbm.at[s].at[g + 1],
                                             sidx[pnext], si[pnext])
                            pltpu.async_copy(dstg_hbm.at[s].at[g + 1],
                                             didx[pnext], si[pnext])
                        pl.when((g >= 1) & (g < NG - 1))(_issue_slab)
            return 0

        lax.fori_loop(0, NGP, pair, 0)
        wait_scat(1)

    @pl.when(c == 0)
    def _():
        run(mag_hbm, 0, lambda g_, e_: g_ * e_)

    @pl.when(c == 1)
    def _():
        run(phase_hbm, 1, lambda g_, e_: g_ + e_)

    plsc.subcore_barrier()

    osl = pl.ds(s * ROWS_MAIN, ROWS_MAIN)
    tail = pl.ds(NS * ROWS_MAIN, N - NS * ROWS_MAIN)

    @pl.when(c == 0)
    def _():
        pltpu.sync_copy(agg_sh.at[osl], aggm_hbm.at[osl])

        @pl.when(s == NS - 1)
        def _():
            pltpu.sync_copy(agg_sh.at[tail], aggm_hbm.at[tail])

    @pl.when(c == 1)
    def _():
        pltpu.sync_copy(agg_sh.at[osl], aggp_hbm.at[osl])

        @pl.when(s == NS - 1)
        def _():
            pltpu.sync_copy(agg_sh.at[tail], aggp_hbm.at[tail])


def _sc_aggregate(mag, phase, ev, srcg, dstg):
    mesh = plsc.VectorSubcoreMesh(core_axis_name="c", subcore_axis_name="s",
                                  num_cores=NC, num_subcores=NS)
    call = pl.kernel(
        _sc_body,
        out_type=(
            jax.ShapeDtypeStruct((N, D), jnp.float32),
            jax.ShapeDtypeStruct((N, D), jnp.float32),
        ),
        mesh=mesh,
        scratch_types=[
            pltpu.VMEM_SHARED((NAGG, D), jnp.float32),
            pltpu.VMEM((G, CHUNK), jnp.int32),
            pltpu.VMEM((G, CHUNK), jnp.int32),
            pltpu.VMEM((G, CHUNK), jnp.int32),
            pltpu.VMEM((G, CHUNK), jnp.int32),
            pltpu.VMEM((CHUNK, D), jnp.float32),
            pltpu.VMEM((CHUNK, D), jnp.float32),
            pltpu.VMEM((CHUNK, D), jnp.float32),
            pltpu.VMEM((CHUNK, D), jnp.float32),
            pltpu.SemaphoreType.DMA,
            pltpu.SemaphoreType.DMA,
            pltpu.SemaphoreType.DMA,
            pltpu.SemaphoreType.DMA,
            pltpu.SemaphoreType.DMA,
            pltpu.SemaphoreType.DMA,
            pltpu.SemaphoreType.DMA,
            pltpu.SemaphoreType.DMA,
        ],
    )
    return call(mag, phase, ev, srcg, dstg)


_BN = 2000


def _node_body(mag_ref, phase_ref, aggm_ref, aggp_ref,
               wg1_ref, wg2_ref, bg_ref, gamma_ref, beta_ref,
               outm_ref, outp_ref):
    mag = mag_ref[...]
    aggm = aggm_ref[...]
    gate = jax.nn.sigmoid(mag @ wg1_ref[...] + aggm @ wg2_ref[...]
                          + bg_ref[...])
    x = mag + gate * aggm
    mu = jnp.mean(x, axis=-1, keepdims=True)
    xc = x - mu
    var = jnp.mean(xc * xc, axis=-1, keepdims=True)
    outm_ref[...] = xc * lax.rsqrt(var + 1e-5) * gamma_ref[...] + beta_ref[...]
    outp_ref[...] = phase_ref[...] + gate * aggp_ref[...]


def _node_update(mag, phase, aggm, aggp, wg, bg, gamma, beta):
    grid = N // _BN
    full = lambda shape: pl.BlockSpec(shape, lambda i: (0,) * len(shape))
    blk = pl.BlockSpec((_BN, D), lambda i: (i, 0))
    return pl.pallas_call(
        _node_body,
        grid=(grid,),
        in_specs=[blk, blk, blk, blk,
                  full((D, D)), full((D, D)),
                  full((1, D)), full((1, D)), full((1, D))],
        out_specs=(blk, blk),
        out_shape=(jax.ShapeDtypeStruct((N, D), jnp.float32),
                   jax.ShapeDtypeStruct((N, D), jnp.float32)),
    )(mag, phase, aggm, aggp, wg[:D], wg[D:],
      bg.reshape(1, D), gamma.reshape(1, D), beta.reshape(1, D))


def kernel(mag, phase, edge_index, rbf, W1m, b1m, W2m, b2m,
           W1p, b1p, W2p, b2p, phase_scale, Wg, bg, gamma, beta):
    pad_n = E2 - E
    pe = jnp.arange(pad_n, dtype=jnp.int32)
    src_p = jnp.concatenate([edge_index[0], (pe * 37) % N])
    dst_p = jnp.concatenate([edge_index[1], N + (pe % PAD_ROWS)])
    srcg = src_p.reshape(NS, NG, G, CHUNK)
    dstg = dst_p.reshape(NS, NG, G, CHUNK)
    ev = _edge_mlp(rbf, W1m, b1m, W2m, b2m, W1p, b1p, W2p, b2p,
                   phase_scale)
    aggm, aggp = _sc_aggregate(mag, phase, ev, srcg, dstg)
    return _node_update(mag, phase, aggm, aggp, Wg, bg, gamma, beta)
